# baseline (speedup 1.0000x reference)
import jax
import jax.numpy as jnp
from jax import lax
from jax.experimental import pallas as pl
from jax.experimental.pallas import tpu as pltpu
from jax.experimental.pallas import tpu_sc as plsc

_B, _H, _W, _D = 1024, 16, 16, 32
_P = _H * _W
_C = 2 * _D
_NC, _NS, _L = 2, 16, 16
_NW = _NC * _NS
_PPW = _P // _NW
_NB16 = _B // _L
_CC = 4


def _sc_body(img_hbm, ttt_hbm, out_hbm, ttt_v, img_v, cidx_v, ov0, ov1,
             s_in, s_o0, s_o1):
    wid = lax.axis_index("s") * _NC + lax.axis_index("c")
    pltpu.async_copy(img_hbm.at[wid], img_v, s_in)
    pltpu.sync_copy(ttt_hbm, ttt_v)
    pltpu.make_async_copy(img_hbm.at[wid], img_v, s_in).wait()

    for p in range(_PPW):
        @plsc.parallel_loop(0, _NB16, unroll=8)
        def _build(b16):
            base = b16 * _L
            i0 = img_v[2 * p, pl.ds(base, _L)]
            i1 = img_v[2 * p + 1, pl.ds(base, _L)]
            cidx_v[p, pl.ds(base, _L)] = (i0 << 4) + i1

    ovs = (ov0, ov1)
    sos = (s_o0, s_o1)
    zero16 = lax.iota(jnp.int32, _L) * 0

    def cc_body(t2, carry):
        for ph in range(2):
            cc = t2 * 2 + ph
            c0 = cc * _CC
            ov = ovs[ph]

            @pl.when(t2 > 0)
            def _drain():
                pltpu.make_async_copy(
                    ov, out_hbm.at[pl.ds(0, _CC), pl.ds(wid, 1)],
                    sos[ph]).wait()

            coffv = [zero16 + (c0 + c4) * _P for c4 in range(_CC)]

            @plsc.parallel_loop(0, _NB16, unroll=32)
            def _gather(b16):
                base = b16 * _L
                cv = [cidx_v[p, pl.ds(base, _L)] for p in range(_PPW)]
                for c4 in range(_CC):
                    for p in range(_PPW):
                        ov[c4, 0, p, pl.ds(base, _L)] = plsc.load_gather(
                            ttt_v, [cv[p] + coffv[c4]])

            pltpu.async_copy(
                ov, out_hbm.at[pl.ds(c0, _CC), pl.ds(wid, 1)], sos[ph])
        return carry

    lax.fori_loop(0, _C // _CC // 2, cc_body, 0)
    for ph in range(2):
        pltpu.make_async_copy(
            ovs[ph], out_hbm.at[pl.ds(0, _CC), pl.ds(wid, 1)],
            sos[ph]).wait()


def kernel(img, entity_table, color_table):
    img_t = jnp.transpose(img, (1, 2, 3, 0)).reshape(_NW, 2 * _PPW, _B)
    ttt = jnp.concatenate([
        jnp.repeat(entity_table[:16].T, 16, axis=1),
        jnp.tile(color_table.T, (1, 16)),
    ], axis=0).reshape(_C * _P)

    mesh = plsc.VectorSubcoreMesh(
        core_axis_name="c", subcore_axis_name="s",
        num_cores=_NC, num_subcores=_NS)
    run = pl.kernel(
        _sc_body, mesh=mesh,
        compiler_params=pltpu.CompilerParams(needs_layout_passes=False),
        out_type=jax.ShapeDtypeStruct((_C, _NW, _PPW, _B), jnp.float32),
        scratch_types=[
            pltpu.VMEM((_C * _P,), jnp.float32),
            pltpu.VMEM((2 * _PPW, _B), jnp.int32),
            pltpu.VMEM((_PPW, _B), jnp.int32),
            pltpu.VMEM((_CC, 1, _PPW, _B), jnp.float32),
            pltpu.VMEM((_CC, 1, _PPW, _B), jnp.float32),
            pltpu.SemaphoreType.DMA,
            pltpu.SemaphoreType.DMA,
            pltpu.SemaphoreType.DMA,
        ],
    )
    out = run(img_t, ttt)
    return jnp.transpose(out.reshape(_C, _H, _W, _B), (3, 0, 1, 2))


# speedup vs baseline: 1.7090x; 1.7090x over previous
import jax
import jax.numpy as jnp
from jax import lax
from jax.experimental import pallas as pl
from jax.experimental.pallas import tpu as pltpu
from jax.experimental.pallas import tpu_sc as plsc

_B, _H, _W, _D = 1024, 16, 16, 32
_P = _H * _W
_C = 2 * _D
_NC, _NS, _L = 2, 16, 16
_NW = _NC * _NS
_PPW = _P // _NW
_NB16 = _B // _L
_CC = 4


def _sc_body(img_hbm, ttt_hbm, out_hbm, ttt_v, img_v, ov0, ov1,
             s_in, s_o0, s_o1):
    wid = lax.axis_index("s") * _NC + lax.axis_index("c")
    pltpu.async_copy(img_hbm.at[wid], img_v, s_in)
    pltpu.sync_copy(ttt_hbm, ttt_v)
    pltpu.make_async_copy(img_hbm.at[wid], img_v, s_in).wait()

    ovs = (ov0, ov1)
    sos = (s_o0, s_o1)
    zero16 = lax.iota(jnp.int32, _L) * 0

    def make_body(half):
        def cc_body(t2, carry):
            for ph in range(2):
                cc = t2 * 2 + ph
                c0 = half * (_C // 2) + cc * _CC
                ov = ovs[ph]

                @pl.when((t2 > 0) | (half > 0))
                def _drain():
                    pltpu.make_async_copy(
                        ov, out_hbm.at[pl.ds(0, _CC), pl.ds(wid, 1)],
                        sos[ph]).wait()

                coffv = [zero16 + (c0 + c4) * _L for c4 in range(_CC)]

                @plsc.parallel_loop(0, _NB16, unroll=16)
                def _gather(b16):
                    base = b16 * _L
                    cv = [img_v[2 * p + half, pl.ds(base, _L)]
                          for p in range(_PPW)]
                    for c4 in range(_CC):
                        for p in range(_PPW):
                            ov[c4, 0, p, pl.ds(base, _L)] = plsc.load_gather(
                                ttt_v, [cv[p] + coffv[c4]])

                pltpu.async_copy(
                    ov, out_hbm.at[pl.ds(c0, _CC), pl.ds(wid, 1)], sos[ph])
            return carry
        return cc_body

    nloop = _C // 2 // _CC // 2
    lax.fori_loop(0, nloop, make_body(0), 0)
    lax.fori_loop(0, nloop, make_body(1), 0)
    for ph in range(2):
        pltpu.make_async_copy(
            ovs[ph], out_hbm.at[pl.ds(0, _CC), pl.ds(wid, 1)],
            sos[ph]).wait()


def kernel(img, entity_table, color_table):
    img_t = jnp.transpose(img, (1, 2, 3, 0)).reshape(_NW, 2 * _PPW, _B)
    ttt = jnp.concatenate([entity_table[:16].T, color_table.T],
                          axis=0).reshape(_C * _L)

    mesh = plsc.VectorSubcoreMesh(
        core_axis_name="c", subcore_axis_name="s",
        num_cores=_NC, num_subcores=_NS)
    run = pl.kernel(
        _sc_body, mesh=mesh,
        compiler_params=pltpu.CompilerParams(needs_layout_passes=False),
        out_type=jax.ShapeDtypeStruct((_C, _NW, _PPW, _B), jnp.float32),
        scratch_types=[
            pltpu.VMEM((_C * _L,), jnp.float32),
            pltpu.VMEM((2 * _PPW, _B), jnp.int32),
            pltpu.VMEM((_CC, 1, _PPW, _B), jnp.float32),
            pltpu.VMEM((_CC, 1, _PPW, _B), jnp.float32),
            pltpu.SemaphoreType.DMA,
            pltpu.SemaphoreType.DMA,
            pltpu.SemaphoreType.DMA,
        ],
    )
    out = run(img_t, ttt)
    return jnp.transpose(out.reshape(_C, _H, _W, _B), (3, 0, 1, 2))


# speedup vs baseline: 2.1035x; 1.2308x over previous
import jax
import jax.numpy as jnp
from jax import lax
from jax.experimental import pallas as pl
from jax.experimental.pallas import tpu as pltpu
from jax.experimental.pallas import tpu_sc as plsc

_B, _H, _W, _D = 1024, 16, 16, 32
_P = _H * _W
_C = 2 * _D
_NC, _NS, _L = 2, 16, 16
_NW = _NC * _NS
_PPW = _P // _NW
_NB16 = _B // _L
_CC = 2
_NPH = 4


def _sc_body(img_hbm, ttt_hbm, out_hbm, ttt_v, img_v, ov0, ov1, ov2, ov3,
             s_in, s_o0, s_o1, s_o2, s_o3):
    wid = lax.axis_index("s") * _NC + lax.axis_index("c")
    pltpu.async_copy(img_hbm.at[wid], img_v, s_in)
    pltpu.sync_copy(ttt_hbm, ttt_v)
    pltpu.make_async_copy(img_hbm.at[wid], img_v, s_in).wait()

    ovs = (ov0, ov1, ov2, ov3)
    sos = (s_o0, s_o1, s_o2, s_o3)
    zero16 = lax.iota(jnp.int32, _L) * 0

    def make_body(half):
        def cc_body(t4, carry):
            for ph in range(_NPH):
                cc = t4 * _NPH + ph
                c0 = half * (_C // 2) + cc * _CC
                ov = ovs[ph]

                @pl.when((t4 > 0) | (half > 0))
                def _drain():
                    pltpu.make_async_copy(
                        ov, out_hbm.at[pl.ds(0, _CC), pl.ds(wid, 1)],
                        sos[ph]).wait()

                coffv = [zero16 + (c0 + c4) * _L for c4 in range(_CC)]

                @plsc.parallel_loop(0, _NB16, unroll=8)
                def _gather(b16):
                    base = b16 * _L
                    cv = [img_v[2 * p + half, pl.ds(base, _L)]
                          for p in range(_PPW)]
                    for c4 in range(_CC):
                        for p in range(_PPW):
                            ov[c4, 0, p, pl.ds(base, _L)] = plsc.load_gather(
                                ttt_v, [cv[p] + coffv[c4]])

                pltpu.async_copy(
                    ov, out_hbm.at[pl.ds(c0, _CC), pl.ds(wid, 1)], sos[ph])
            return carry
        return cc_body

    nloop = _C // 2 // _CC // _NPH
    lax.fori_loop(0, nloop, make_body(0), 0)
    lax.fori_loop(0, nloop, make_body(1), 0)
    for ph in range(_NPH):
        pltpu.make_async_copy(
            ovs[ph], out_hbm.at[pl.ds(0, _CC), pl.ds(wid, 1)],
            sos[ph]).wait()


def kernel(img, entity_table, color_table):
    img_t = jnp.transpose(img, (1, 2, 3, 0)).reshape(_NW, 2 * _PPW, _B)
    ttt = jnp.concatenate([entity_table[:16].T, color_table.T],
                          axis=0).reshape(_C * _L)

    mesh = plsc.VectorSubcoreMesh(
        core_axis_name="c", subcore_axis_name="s",
        num_cores=_NC, num_subcores=_NS)
    run = pl.kernel(
        _sc_body, mesh=mesh,
        compiler_params=pltpu.CompilerParams(needs_layout_passes=False),
        out_type=jax.ShapeDtypeStruct((_C, _NW, _PPW, _B), jnp.float32),
        scratch_types=[
            pltpu.VMEM((_C * _L,), jnp.float32),
            pltpu.VMEM((2 * _PPW, _B), jnp.int32),
            pltpu.VMEM((_CC, 1, _PPW, _B), jnp.float32),
            pltpu.VMEM((_CC, 1, _PPW, _B), jnp.float32),
            pltpu.VMEM((_CC, 1, _PPW, _B), jnp.float32),
            pltpu.VMEM((_CC, 1, _PPW, _B), jnp.float32),
            pltpu.SemaphoreType.DMA,
            pltpu.SemaphoreType.DMA,
            pltpu.SemaphoreType.DMA,
            pltpu.SemaphoreType.DMA,
            pltpu.SemaphoreType.DMA,
        ],
    )
    out = run(img_t, ttt)
    return jnp.transpose(out.reshape(_C, _H, _W, _B), (3, 0, 1, 2))
